# baseline (speedup 1.0000x reference)
import jax
import jax.numpy as jnp
from jax import lax
from jax.experimental import pallas as pl
from jax.experimental.pallas import tpu as pltpu
from jax.experimental.pallas import tpu_sc as plsc

_NUM_FEATURES = 256
_NUM_CLUSTERS = 8192
_EPS = 1e-6

_BT = 1024
_BK = 2048

_SC_CORES = 2
_SC_SUBCORES = 16
_SC_WORKERS = _SC_CORES * _SC_SUBCORES


def _assign_body(x_ref, c_ref, xsq_ref, xsum_ref, csq_ref, csum_ref,
                 lab_ref, val_ref, chunk_ref):
    j = pl.program_id(1)
    nj = pl.num_programs(1)

    @pl.when(j == 0)
    def _():
        val_ref[...] = jnp.full_like(val_ref, jnp.inf)
        chunk_ref[...] = jnp.zeros_like(chunk_ref)

    xb = x_ref[...]
    cb = c_ref[...]
    bk = cb.shape[0]

    cross = lax.dot_general(xb, cb, (((1,), (1,)), ((), ())),
                            preferred_element_type=jnp.float32)

    x_sq = xsq_ref[...]
    x_sum = xsum_ref[...]
    run_val = val_ref[...]
    run_chunk = chunk_ref[...]
    for c in range(bk // 128):
        c_sq = csq_ref[:, pl.ds(c * 128, 128)]
        c_sum = csum_ref[:, pl.ds(c * 128, 128)]
        cr = cross[:, c * 128:(c + 1) * 128]
        sq = x_sq + c_sq - 2.0 * cr + (2.0 * _EPS) * (x_sum - c_sum) \
            + _NUM_FEATURES * _EPS * _EPS
        dist = jnp.sqrt(jnp.maximum(sq, 0.0))
        better = dist < run_val
        run_val = jnp.where(better, dist, run_val)
        run_chunk = jnp.where(better, j * (bk // 128) + c, run_chunk)
    val_ref[...] = run_val
    chunk_ref[...] = run_chunk

    @pl.when(j == nj - 1)
    def _():
        lane = lax.broadcasted_iota(jnp.int32, run_chunk.shape, 1)
        gidx = run_chunk * 128 + lane
        rowmin = jnp.min(run_val, axis=1, keepdims=True)
        big = jnp.int32(_NUM_CLUSTERS)
        lab_ref[...] = jnp.min(jnp.where(run_val == rowmin, gidx, big),
                               axis=1, keepdims=True)


def _assign_labels(xf, centroids, x_sq, x_sum, c_sq, c_sum):
    n = xf.shape[0]
    k = centroids.shape[0]
    grid = (n // _BT, k // _BK)
    return pl.pallas_call(
        _assign_body,
        grid=grid,
        in_specs=[
            pl.BlockSpec((_BT, _NUM_FEATURES), lambda t, j: (t, 0)),
            pl.BlockSpec((_BK, _NUM_FEATURES), lambda t, j: (j, 0)),
            pl.BlockSpec((_BT, 1), lambda t, j: (t, 0)),
            pl.BlockSpec((_BT, 1), lambda t, j: (t, 0)),
            pl.BlockSpec((1, _BK), lambda t, j: (0, j)),
            pl.BlockSpec((1, _BK), lambda t, j: (0, j)),
        ],
        out_specs=pl.BlockSpec((_BT, 1), lambda t, j: (t, 0)),
        out_shape=jax.ShapeDtypeStruct((n, 1), jnp.int32),
        scratch_shapes=[
            pltpu.VMEM((_BT, 128), jnp.float32),
            pltpu.VMEM((_BT, 128), jnp.int32),
        ],
        compiler_params=pltpu.CompilerParams(
            dimension_semantics=("arbitrary", "arbitrary"),
        ),
    )(xf, centroids, x_sq, x_sum, c_sq, c_sum)


def _gather_body(table_hbm, idx_hbm, out_hbm, idx_v, rows_v, sem):
    wid = lax.axis_index("s") * _SC_CORES + lax.axis_index("c")
    bpw = idx_v.shape[0]
    base = wid * bpw
    pltpu.sync_copy(idx_hbm.at[pl.ds(base, bpw)], idx_v)
    pltpu.async_copy(table_hbm.at[idx_v], rows_v, sem).wait()
    pltpu.sync_copy(rows_v, out_hbm.at[pl.ds(base, bpw)])


def _gather_rows(centroids, labels):
    n = labels.shape[0]
    bpw = n // _SC_WORKERS
    mesh = plsc.VectorSubcoreMesh(core_axis_name="c", subcore_axis_name="s")
    return pl.kernel(
        _gather_body,
        out_type=jax.ShapeDtypeStruct((n, _NUM_FEATURES), jnp.float32),
        mesh=mesh,
        scratch_types=[
            pltpu.VMEM((bpw,), jnp.int32),
            pltpu.VMEM((bpw, _NUM_FEATURES), jnp.float32),
            pltpu.SemaphoreType.DMA,
        ],
    )(centroids, labels)


def kernel(x, centroids):
    batch_shape = x.shape[:-1]
    nf = centroids.shape[-1]
    xf = x.reshape(-1, nf)
    x_sq = jnp.sum(xf * xf, axis=-1, keepdims=True)
    x_sum = jnp.sum(xf, axis=-1, keepdims=True)
    c_sq = jnp.sum(centroids * centroids, axis=-1)[None, :]
    c_sum = jnp.sum(centroids, axis=-1)[None, :]
    labels2d = _assign_labels(xf, centroids, x_sq, x_sum, c_sq, c_sum)
    labels = labels2d.reshape(-1)
    assigned = _gather_rows(centroids, labels)
    return labels.reshape(batch_shape), assigned.reshape(batch_shape + (nf,))


# speedup vs baseline: 1.3361x; 1.3361x over previous
import jax
import jax.numpy as jnp
from jax import lax
from jax.experimental import pallas as pl
from jax.experimental.pallas import tpu as pltpu
from jax.experimental.pallas import tpu_sc as plsc

_NUM_FEATURES = 256
_NUM_CLUSTERS = 8192
_EPS = 1e-6

_BT = 1024
_BK = 2048

_SC_CORES = 2
_SC_SUBCORES = 16
_SC_WORKERS = _SC_CORES * _SC_SUBCORES


def _assign_body(x_ref, c_ref, xsq_ref, xsum_ref, csq_ref, csum_ref,
                 lab_ref, val_ref, chunk_ref):
    j = pl.program_id(1)
    nj = pl.num_programs(1)

    @pl.when(j == 0)
    def _():
        val_ref[...] = jnp.full_like(val_ref, jnp.inf)
        chunk_ref[...] = jnp.zeros_like(chunk_ref)

    xb = x_ref[...]
    cb = c_ref[...]
    bk = cb.shape[0]

    cross = lax.dot_general(xb, cb, (((1,), (1,)), ((), ())),
                            preferred_element_type=jnp.float32)

    x_sq = xsq_ref[...]
    x_sum = xsum_ref[...]
    run_val = val_ref[...]
    run_chunk = chunk_ref[...]
    for c in range(bk // 128):
        c_sq = csq_ref[:, pl.ds(c * 128, 128)]
        c_sum = csum_ref[:, pl.ds(c * 128, 128)]
        cr = cross[:, c * 128:(c + 1) * 128]
        sq = x_sq + c_sq - 2.0 * cr + (2.0 * _EPS) * (x_sum - c_sum) \
            + _NUM_FEATURES * _EPS * _EPS
        better = sq < run_val
        run_val = jnp.where(better, sq, run_val)
        run_chunk = jnp.where(better, j * (bk // 128) + c, run_chunk)
    val_ref[...] = run_val
    chunk_ref[...] = run_chunk

    @pl.when(j == nj - 1)
    def _():
        dist = jnp.sqrt(jnp.maximum(val_ref[...], 0.0))
        run_chunk = chunk_ref[...]
        lane = lax.broadcasted_iota(jnp.int32, run_chunk.shape, 1)
        gidx = run_chunk * 128 + lane
        rowmin = jnp.min(dist, axis=1, keepdims=True)
        big = jnp.int32(_NUM_CLUSTERS)
        lab_ref[...] = jnp.min(jnp.where(dist == rowmin, gidx, big),
                               axis=1, keepdims=True)


def _assign_labels(xf, centroids, x_sq, x_sum, c_sq, c_sum):
    n = xf.shape[0]
    k = centroids.shape[0]
    grid = (n // _BT, k // _BK)
    return pl.pallas_call(
        _assign_body,
        grid=grid,
        in_specs=[
            pl.BlockSpec((_BT, _NUM_FEATURES), lambda t, j: (t, 0)),
            pl.BlockSpec((_BK, _NUM_FEATURES), lambda t, j: (j, 0)),
            pl.BlockSpec((_BT, 1), lambda t, j: (t, 0)),
            pl.BlockSpec((_BT, 1), lambda t, j: (t, 0)),
            pl.BlockSpec((1, _BK), lambda t, j: (0, j)),
            pl.BlockSpec((1, _BK), lambda t, j: (0, j)),
        ],
        out_specs=pl.BlockSpec((_BT, 1), lambda t, j: (t, 0)),
        out_shape=jax.ShapeDtypeStruct((n, 1), jnp.int32),
        scratch_shapes=[
            pltpu.VMEM((_BT, 128), jnp.float32),
            pltpu.VMEM((_BT, 128), jnp.int32),
        ],
        compiler_params=pltpu.CompilerParams(
            dimension_semantics=("arbitrary", "arbitrary"),
        ),
    )(xf, centroids, x_sq, x_sum, c_sq, c_sum)


def _gather_body(table_hbm, idx_hbm, out_hbm, idx_v, rows_v, sem):
    wid = lax.axis_index("s") * _SC_CORES + lax.axis_index("c")
    bpw = idx_v.shape[0]
    base = wid * bpw
    pltpu.sync_copy(idx_hbm.at[pl.ds(base, bpw)], idx_v)
    pltpu.async_copy(table_hbm.at[idx_v], rows_v, sem).wait()
    pltpu.sync_copy(rows_v, out_hbm.at[pl.ds(base, bpw)])


def _gather_rows(centroids, labels):
    n = labels.shape[0]
    bpw = n // _SC_WORKERS
    mesh = plsc.VectorSubcoreMesh(core_axis_name="c", subcore_axis_name="s")
    return pl.kernel(
        _gather_body,
        out_type=jax.ShapeDtypeStruct((n, _NUM_FEATURES), jnp.float32),
        mesh=mesh,
        scratch_types=[
            pltpu.VMEM((bpw,), jnp.int32),
            pltpu.VMEM((bpw, _NUM_FEATURES), jnp.float32),
            pltpu.SemaphoreType.DMA,
        ],
    )(centroids, labels)


def kernel(x, centroids):
    batch_shape = x.shape[:-1]
    nf = centroids.shape[-1]
    xf = x.reshape(-1, nf)
    x_sq = jnp.sum(xf * xf, axis=-1, keepdims=True)
    x_sum = jnp.sum(xf, axis=-1, keepdims=True)
    c_sq = jnp.sum(centroids * centroids, axis=-1)[None, :]
    c_sum = jnp.sum(centroids, axis=-1)[None, :]
    labels2d = _assign_labels(xf, centroids, x_sq, x_sum, c_sq, c_sum)
    labels = labels2d.reshape(-1)
    assigned = _gather_rows(centroids, labels)
    return labels.reshape(batch_shape), assigned.reshape(batch_shape + (nf,))


# speedup vs baseline: 1.3802x; 1.0330x over previous
import jax
import jax.numpy as jnp
from jax import lax
from jax.experimental import pallas as pl
from jax.experimental.pallas import tpu as pltpu
from jax.experimental.pallas import tpu_sc as plsc

_NUM_FEATURES = 256
_NUM_CLUSTERS = 8192
_EPS = 1e-6

_BT = 2048
_BK = 2048

_SC_CORES = 2
_SC_SUBCORES = 16
_SC_WORKERS = _SC_CORES * _SC_SUBCORES


def _assign_body(x_ref, c_ref, xsq_ref, xsum_ref, csq_ref, csum_ref,
                 lab_ref, val_ref, chunk_ref):
    j = pl.program_id(1)
    nj = pl.num_programs(1)

    @pl.when(j == 0)
    def _():
        val_ref[...] = jnp.full_like(val_ref, jnp.inf)
        chunk_ref[...] = jnp.zeros_like(chunk_ref)

    xb = x_ref[...]
    cb = c_ref[...]
    bk = cb.shape[0]

    cross = lax.dot_general(xb, cb, (((1,), (1,)), ((), ())),
                            preferred_element_type=jnp.float32)

    x_sq = xsq_ref[...]
    x_sum = xsum_ref[...]
    run_val = val_ref[...]
    run_chunk = chunk_ref[...]
    for c in range(bk // 128):
        c_sq = csq_ref[:, pl.ds(c * 128, 128)]
        c_sum = csum_ref[:, pl.ds(c * 128, 128)]
        cr = cross[:, c * 128:(c + 1) * 128]
        sq = x_sq + c_sq - 2.0 * cr + (2.0 * _EPS) * (x_sum - c_sum) \
            + _NUM_FEATURES * _EPS * _EPS
        better = sq < run_val
        run_val = jnp.where(better, sq, run_val)
        run_chunk = jnp.where(better, j * (bk // 128) + c, run_chunk)
    val_ref[...] = run_val
    chunk_ref[...] = run_chunk

    @pl.when(j == nj - 1)
    def _():
        dist = jnp.sqrt(jnp.maximum(val_ref[...], 0.0))
        run_chunk = chunk_ref[...]
        lane = lax.broadcasted_iota(jnp.int32, run_chunk.shape, 1)
        gidx = run_chunk * 128 + lane
        rowmin = jnp.min(dist, axis=1, keepdims=True)
        big = jnp.int32(_NUM_CLUSTERS)
        lab_ref[...] = jnp.min(jnp.where(dist == rowmin, gidx, big),
                               axis=1, keepdims=True)


def _assign_labels(xf, centroids, x_sq, x_sum, c_sq, c_sum):
    n = xf.shape[0]
    k = centroids.shape[0]
    grid = (n // _BT, k // _BK)
    return pl.pallas_call(
        _assign_body,
        grid=grid,
        in_specs=[
            pl.BlockSpec((_BT, _NUM_FEATURES), lambda t, j: (t, 0)),
            pl.BlockSpec((_BK, _NUM_FEATURES), lambda t, j: (j, 0)),
            pl.BlockSpec((_BT, 1), lambda t, j: (t, 0)),
            pl.BlockSpec((_BT, 1), lambda t, j: (t, 0)),
            pl.BlockSpec((1, _BK), lambda t, j: (0, j)),
            pl.BlockSpec((1, _BK), lambda t, j: (0, j)),
        ],
        out_specs=pl.BlockSpec((_BT, 1), lambda t, j: (t, 0)),
        out_shape=jax.ShapeDtypeStruct((n, 1), jnp.int32),
        scratch_shapes=[
            pltpu.VMEM((_BT, 128), jnp.float32),
            pltpu.VMEM((_BT, 128), jnp.int32),
        ],
        compiler_params=pltpu.CompilerParams(
            dimension_semantics=("arbitrary", "arbitrary"),
        ),
    )(xf, centroids, x_sq, x_sum, c_sq, c_sum)


def _gather_body(table_hbm, idx_hbm, out_hbm, idx_v, rows_v, sem):
    wid = lax.axis_index("s") * _SC_CORES + lax.axis_index("c")
    bpw = idx_v.shape[0]
    base = wid * bpw
    pltpu.sync_copy(idx_hbm.at[pl.ds(base, bpw)], idx_v)
    pltpu.async_copy(table_hbm.at[idx_v], rows_v, sem).wait()
    pltpu.sync_copy(rows_v, out_hbm.at[pl.ds(base, bpw)])


def _gather_rows(centroids, labels):
    n = labels.shape[0]
    bpw = n // _SC_WORKERS
    mesh = plsc.VectorSubcoreMesh(core_axis_name="c", subcore_axis_name="s")
    return pl.kernel(
        _gather_body,
        out_type=jax.ShapeDtypeStruct((n, _NUM_FEATURES), jnp.float32),
        mesh=mesh,
        scratch_types=[
            pltpu.VMEM((bpw,), jnp.int32),
            pltpu.VMEM((bpw, _NUM_FEATURES), jnp.float32),
            pltpu.SemaphoreType.DMA,
        ],
    )(centroids, labels)


def kernel(x, centroids):
    batch_shape = x.shape[:-1]
    nf = centroids.shape[-1]
    xf = x.reshape(-1, nf)
    x_sq = jnp.sum(xf * xf, axis=-1, keepdims=True)
    x_sum = jnp.sum(xf, axis=-1, keepdims=True)
    c_sq = jnp.sum(centroids * centroids, axis=-1)[None, :]
    c_sum = jnp.sum(centroids, axis=-1)[None, :]
    labels2d = _assign_labels(xf, centroids, x_sq, x_sum, c_sq, c_sum)
    labels = labels2d.reshape(-1)
    assigned = _gather_rows(centroids, labels)
    return labels.reshape(batch_shape), assigned.reshape(batch_shape + (nf,))
